# Initial kernel scaffold; baseline (speedup 1.0000x reference)
import functools

import jax
import jax.numpy as jnp
from jax import lax
from jax.experimental import pallas as pl
from jax.experimental.pallas import tpu as pltpu
from jax.experimental.pallas import tpu_sc as plsc

N = 10000
E = 320000
HID = 64
NODE_IN = 5
EDGE_DIM = 3

NC = 2
NS = 16
NW = NC * NS
PW = E // NW
K = 80
NCHUNK = PW // K
NPAD = 10240
RPT = NPAD // NS

_mesh = plsc.VectorSubcoreMesh(
    core_axis_name="c", subcore_axis_name="s", num_cores=NC, num_subcores=NS)

f32 = jnp.float32


def _gather_body(x_hbm, src_hbm, dst_hbm, gs_hbm, gd_hbm,
                 src_v, dst_v, bufs, bufd):
    c = lax.axis_index("c")
    s = lax.axis_index("s")
    wid = s * NC + c
    base = wid * PW
    pltpu.sync_copy(src_hbm.at[pl.ds(base, PW)], src_v)
    pltpu.sync_copy(dst_hbm.at[pl.ds(base, PW)], dst_v)

    def body(j, carry):
        off = j * K
        pltpu.sync_copy(x_hbm.at[src_v.at[pl.ds(off, K)]], bufs)
        pltpu.sync_copy(x_hbm.at[dst_v.at[pl.ds(off, K)]], bufd)
        pltpu.sync_copy(bufs, gs_hbm.at[pl.ds(base + off, K)])
        pltpu.sync_copy(bufd, gd_hbm.at[pl.ds(base + off, K)])
        return carry

    lax.fori_loop(0, NCHUNK, body, 0)


_gather = pl.kernel(
    _gather_body,
    out_type=[jax.ShapeDtypeStruct((E, HID), f32),
              jax.ShapeDtypeStruct((E, HID), f32)],
    mesh=_mesh,
    scratch_types=[
        pltpu.VMEM((PW,), jnp.int32),
        pltpu.VMEM((PW,), jnp.int32),
        pltpu.VMEM((K, HID), f32),
        pltpu.VMEM((K, HID), f32),
    ],
)


def _scatter_body(m_hbm, dst2_hbm, agg_hbm, dst_v, mbuf, zbuf, agg_sh):
    c = lax.axis_index("c")
    s = lax.axis_index("s")
    wid = s * NC + c
    base = wid * PW

    def zbody(i, carry):
        zbuf[pl.ds(i * 16, 16)] = jnp.zeros((16,), f32)
        return carry
    lax.fori_loop(0, (RPT * HID) // 16, zbody, 0)
    pltpu.sync_copy(zbuf.reshape(RPT, HID), agg_sh.at[pl.ds(s * RPT, RPT)])
    plsc.subcore_barrier()

    pltpu.sync_copy(dst2_hbm.at[pl.ds(wid * NCHUNK, NCHUNK)], dst_v)

    def body(j, carry):
        pltpu.sync_copy(m_hbm.at[pl.ds(base + j * K, K)], mbuf)
        pltpu.sync_copy(mbuf, agg_sh.at[dst_v.at[j]], add=True)
        return carry
    lax.fori_loop(0, NCHUNK, body, 0)
    plsc.subcore_barrier()

    pltpu.sync_copy(agg_sh.at[pl.ds(s * RPT, RPT)], zbuf.reshape(RPT, HID))
    pltpu.sync_copy(zbuf.reshape(RPT, HID), agg_hbm.at[c, pl.ds(s * RPT, RPT)])


_scatter = pl.kernel(
    _scatter_body,
    out_type=jax.ShapeDtypeStruct((NC, NPAD, HID), f32),
    mesh=_mesh,
    scratch_types=[
        pltpu.VMEM((NCHUNK, K), jnp.int32),
        pltpu.VMEM((K, HID), f32),
        pltpu.VMEM((RPT * HID,), f32),
        pltpu.VMEM_SHARED((NPAD, HID), f32),
    ],
)


def _encoder_body(h_ref, w_ref, b_ref, out_ref):
    out_ref[...] = h_ref[...] @ w_ref[...] + b_ref[...]


def _edge_mlp_body(gs_ref, gd_ref, attr_ref, ws_ref, wd_ref, we_ref,
                   b1_ref, w2_ref, b2_ref, out_ref):
    z = (gs_ref[...] @ ws_ref[...] + gd_ref[...] @ wd_ref[...]
         + attr_ref[...] @ we_ref[...] + b1_ref[...])
    m1 = jnp.maximum(z, 0.0)
    out_ref[...] = jnp.maximum(m1 @ w2_ref[...] + b2_ref[...], 0.0)


def _update_body(x_ref, agg_ref, w1x_ref, w1a_ref, b1_ref, w2_ref, b2_ref,
                 out_ref):
    agg = agg_ref[0] + agg_ref[1]
    z = jnp.maximum(
        x_ref[...] @ w1x_ref[...] + agg @ w1a_ref[...] + b1_ref[...], 0.0)
    out_ref[...] = x_ref[...] + (z @ w2_ref[...] + b2_ref[...])


def _decoder_body(x_ref, w1_ref, b1_ref, w2_ref, b2_ref, out_ref):
    d = jnp.maximum(x_ref[...] @ w1_ref[...] + b1_ref[...], 0.0)
    out_ref[...] = d @ w2_ref[...] + b2_ref[...]


EB = 2000


def _full(shape):
    return pl.BlockSpec(shape, lambda *_: tuple(0 for _ in shape))


def _encoder(h, w, b):
    return pl.pallas_call(
        _encoder_body,
        out_shape=jax.ShapeDtypeStruct((N, HID), f32),
        in_specs=[_full((N, NODE_IN)), _full((NODE_IN, HID)), _full((1, HID))],
        out_specs=_full((N, HID)),
    )(h, w, b)


def _edge_mlp(gs, gd, attr, ws, wd, we, b1, w2, b2):
    grid = (E // EB,)
    eb = pl.BlockSpec((EB, HID), lambda i: (i, 0))
    ea = pl.BlockSpec((EB, EDGE_DIM), lambda i: (i, 0))
    wspec = pl.BlockSpec((HID, HID), lambda i: (0, 0))
    wespec = pl.BlockSpec((EDGE_DIM, HID), lambda i: (0, 0))
    bspec = pl.BlockSpec((1, HID), lambda i: (0, 0))
    return pl.pallas_call(
        _edge_mlp_body,
        grid=grid,
        out_shape=jax.ShapeDtypeStruct((E, HID), f32),
        in_specs=[eb, eb, ea, wspec, wspec, wespec, bspec, wspec, bspec],
        out_specs=eb,
    )(gs, gd, attr, ws, wd, we, b1, w2, b2)


def _update(x, agg, w1x, w1a, b1, w2, b2):
    return pl.pallas_call(
        _update_body,
        out_shape=jax.ShapeDtypeStruct((N, HID), f32),
        in_specs=[_full((N, HID)), _full((NC, N, HID)), _full((HID, HID)),
                  _full((HID, HID)), _full((1, HID)), _full((HID, HID)),
                  _full((1, HID))],
        out_specs=_full((N, HID)),
    )(x, agg, w1x, w1a, b1, w2, b2)


def _decoder(x, w1, b1, w2, b2):
    return pl.pallas_call(
        _decoder_body,
        out_shape=jax.ShapeDtypeStruct((N, 1), f32),
        in_specs=[_full((N, HID)), _full((HID, HID)), _full((1, HID)),
                  _full((HID, 1)), _full((1, 1))],
        out_specs=_full((N, 1)),
    )(x, w1, b1, w2, b2)


def kernel(h, edge_index, edge_attr, params):
    src = edge_index[0]
    dst = edge_index[1]
    dst2 = dst.reshape(E // K, K)

    x = _encoder(h, params["enc_w"], params["enc_b"].reshape(1, HID))
    for p in params["layers"]:
        ws = p["m_w1"][:HID]
        wd = p["m_w1"][HID:2 * HID]
        we = p["m_w1"][2 * HID:]
        gs, gd = _gather(x, src, dst)
        m = _edge_mlp(gs, gd, edge_attr, ws, wd, we,
                      p["m_b1"].reshape(1, HID), p["m_w2"],
                      p["m_b2"].reshape(1, HID))
        aggp = _scatter(m, dst2)
        agg = aggp[:, :N, :]
        x = _update(x, agg, p["u_w1"][:HID], p["u_w1"][HID:],
                    p["u_b1"].reshape(1, HID), p["u_w2"],
                    p["u_b2"].reshape(1, HID))
    psi = _decoder(x, params["dec_w1"], params["dec_b1"].reshape(1, HID),
                   params["dec_w2"], params["dec_b2"].reshape(1, 1))
    return psi


# speedup vs baseline: 1.7667x; 1.7667x over previous
import functools

import jax
import jax.numpy as jnp
from jax import lax
from jax.experimental import pallas as pl
from jax.experimental.pallas import tpu as pltpu
from jax.experimental.pallas import tpu_sc as plsc

N = 10000
E = 320000
HID = 64
NODE_IN = 5
EDGE_DIM = 3

NC = 2
NS = 16
NW = NC * NS
PW = E // NW
K = 80
NCHUNK = PW // K
NPAD = 10240
RPT = NPAD // NS

_mesh = plsc.VectorSubcoreMesh(
    core_axis_name="c", subcore_axis_name="s", num_cores=NC, num_subcores=NS)

_sc_params = pltpu.CompilerParams(use_tc_tiling_on_sc=False)

f32 = jnp.float32


def _gather_body(x_hbm, src_hbm, dst_hbm, gs_hbm, gd_hbm,
                 src_v, dst_v, bufs, bufd):
    c = lax.axis_index("c")
    s = lax.axis_index("s")
    wid = s * NC + c
    base = wid * PW
    pltpu.sync_copy(src_hbm.at[pl.ds(base, PW)], src_v)
    pltpu.sync_copy(dst_hbm.at[pl.ds(base, PW)], dst_v)

    def body(j, carry):
        off = j * K
        pltpu.sync_copy(x_hbm.at[src_v.at[pl.ds(off, K)]], bufs)
        pltpu.sync_copy(x_hbm.at[dst_v.at[pl.ds(off, K)]], bufd)
        pltpu.sync_copy(bufs, gs_hbm.at[pl.ds(base + off, K)])
        pltpu.sync_copy(bufd, gd_hbm.at[pl.ds(base + off, K)])
        return carry

    lax.fori_loop(0, NCHUNK, body, 0)


_gather = pl.kernel(
    _gather_body,
    out_type=[jax.ShapeDtypeStruct((E, HID), f32),
              jax.ShapeDtypeStruct((E, HID), f32)],
    mesh=_mesh,
    scratch_types=[
        pltpu.VMEM((PW,), jnp.int32),
        pltpu.VMEM((PW,), jnp.int32),
        pltpu.VMEM((K, HID), f32),
        pltpu.VMEM((K, HID), f32),
    ],
    compiler_params=_sc_params,
)


def _scatter_body(m_hbm, dst2_hbm, agg_hbm, dst_v, mbuf, zbuf, agg_sh):
    c = lax.axis_index("c")
    s = lax.axis_index("s")
    wid = s * NC + c
    base = wid * PW

    def zbody(r, carry):
        for l in range(HID // 16):
            zbuf[r, pl.ds(l * 16, 16)] = jnp.zeros((16,), f32)
        return carry
    lax.fori_loop(0, RPT, zbody, 0)
    pltpu.sync_copy(zbuf, agg_sh.at[pl.ds(s * RPT, RPT)])
    plsc.subcore_barrier()

    pltpu.sync_copy(dst2_hbm.at[pl.ds(wid * NCHUNK, NCHUNK)], dst_v)

    def body(j, carry):
        pltpu.sync_copy(m_hbm.at[pl.ds(base + j * K, K)], mbuf)
        pltpu.sync_copy(mbuf, agg_sh.at[dst_v.at[j]], add=True)
        return carry
    lax.fori_loop(0, NCHUNK, body, 0)
    plsc.subcore_barrier()

    pltpu.sync_copy(agg_sh.at[pl.ds(s * RPT, RPT)], zbuf)
    pltpu.sync_copy(zbuf, agg_hbm.at[c, pl.ds(s * RPT, RPT)])


_scatter = pl.kernel(
    _scatter_body,
    out_type=jax.ShapeDtypeStruct((NC, NPAD, HID), f32),
    mesh=_mesh,
    scratch_types=[
        pltpu.VMEM((NCHUNK, K), jnp.int32),
        pltpu.VMEM((K, HID), f32),
        pltpu.VMEM((RPT, HID), f32),
        pltpu.VMEM_SHARED((NPAD, HID), f32),
    ],
    compiler_params=_sc_params,
)


def _encoder_body(h_ref, w_ref, b_ref, out_ref):
    out_ref[...] = h_ref[...] @ w_ref[...] + b_ref[...]


def _edge_mlp_body(gs_ref, gd_ref, attr_ref, ws_ref, wd_ref, we_ref,
                   b1_ref, w2_ref, b2_ref, out_ref):
    z = (gs_ref[...] @ ws_ref[...] + gd_ref[...] @ wd_ref[...]
         + attr_ref[...] @ we_ref[...] + b1_ref[...])
    m1 = jnp.maximum(z, 0.0)
    out_ref[...] = jnp.maximum(m1 @ w2_ref[...] + b2_ref[...], 0.0)


def _update_body(x_ref, agg_ref, w1x_ref, w1a_ref, b1_ref, w2_ref, b2_ref,
                 out_ref):
    agg = agg_ref[0] + agg_ref[1]
    z = jnp.maximum(
        x_ref[...] @ w1x_ref[...] + agg @ w1a_ref[...] + b1_ref[...], 0.0)
    out_ref[...] = x_ref[...] + (z @ w2_ref[...] + b2_ref[...])


def _decoder_body(x_ref, w1_ref, b1_ref, w2_ref, b2_ref, out_ref):
    d = jnp.maximum(x_ref[...] @ w1_ref[...] + b1_ref[...], 0.0)
    out_ref[...] = d @ w2_ref[...] + b2_ref[...]


EB = 2000


def _full(shape):
    return pl.BlockSpec(shape, lambda *_: tuple(0 for _ in shape))


def _encoder(h, w, b):
    return pl.pallas_call(
        _encoder_body,
        out_shape=jax.ShapeDtypeStruct((N, HID), f32),
        in_specs=[_full((N, NODE_IN)), _full((NODE_IN, HID)), _full((1, HID))],
        out_specs=_full((N, HID)),
    )(h, w, b)


def _edge_mlp(gs, gd, attr, ws, wd, we, b1, w2, b2):
    grid = (E // EB,)
    eb = pl.BlockSpec((EB, HID), lambda i: (i, 0))
    ea = pl.BlockSpec((EB, EDGE_DIM), lambda i: (i, 0))
    wspec = pl.BlockSpec((HID, HID), lambda i: (0, 0))
    wespec = pl.BlockSpec((EDGE_DIM, HID), lambda i: (0, 0))
    bspec = pl.BlockSpec((1, HID), lambda i: (0, 0))
    return pl.pallas_call(
        _edge_mlp_body,
        grid=grid,
        out_shape=jax.ShapeDtypeStruct((E, HID), f32),
        in_specs=[eb, eb, ea, wspec, wspec, wespec, bspec, wspec, bspec],
        out_specs=eb,
    )(gs, gd, attr, ws, wd, we, b1, w2, b2)


def _update(x, agg, w1x, w1a, b1, w2, b2):
    return pl.pallas_call(
        _update_body,
        out_shape=jax.ShapeDtypeStruct((N, HID), f32),
        in_specs=[_full((N, HID)), _full((NC, N, HID)), _full((HID, HID)),
                  _full((HID, HID)), _full((1, HID)), _full((HID, HID)),
                  _full((1, HID))],
        out_specs=_full((N, HID)),
    )(x, agg, w1x, w1a, b1, w2, b2)


def _decoder(x, w1, b1, w2, b2):
    return pl.pallas_call(
        _decoder_body,
        out_shape=jax.ShapeDtypeStruct((N, 1), f32),
        in_specs=[_full((N, HID)), _full((HID, HID)), _full((1, HID)),
                  _full((HID, 1)), _full((1, 1))],
        out_specs=_full((N, 1)),
    )(x, w1, b1, w2, b2)


def kernel(h, edge_index, edge_attr, params):
    src = edge_index[0]
    dst = edge_index[1]
    dst2 = dst.reshape(E // K, K)

    x = _encoder(h, params["enc_w"], params["enc_b"].reshape(1, HID))
    for p in params["layers"]:
        ws = p["m_w1"][:HID]
        wd = p["m_w1"][HID:2 * HID]
        we = p["m_w1"][2 * HID:]
        gs, gd = _gather(x, src, dst)
        m = _edge_mlp(gs, gd, edge_attr, ws, wd, we,
                      p["m_b1"].reshape(1, HID), p["m_w2"],
                      p["m_b2"].reshape(1, HID))
        aggp = _scatter(m, dst2)
        agg = aggp[:, :N, :]
        x = _update(x, agg, p["u_w1"][:HID], p["u_w1"][HID:],
                    p["u_b1"].reshape(1, HID), p["u_w2"],
                    p["u_b2"].reshape(1, HID))
    psi = _decoder(x, params["dec_w1"], params["dec_b1"].reshape(1, HID),
                   params["dec_w2"], params["dec_b2"].reshape(1, 1))
    return psi


# speedup vs baseline: 2.2196x; 1.2564x over previous
import functools

import jax
import jax.numpy as jnp
from jax import lax
from jax.experimental import pallas as pl
from jax.experimental.pallas import tpu as pltpu
from jax.experimental.pallas import tpu_sc as plsc

N = 10000
E = 320000
HID = 64
NODE_IN = 5
EDGE_DIM = 3

NC = 2
NS = 16
NW = NC * NS
PW = E // NW
K = 80
NCHUNK = PW // K
NPAD = 10240
RPT = NPAD // NS

_mesh = plsc.VectorSubcoreMesh(
    core_axis_name="c", subcore_axis_name="s", num_cores=NC, num_subcores=NS)

_sc_params = pltpu.CompilerParams(use_tc_tiling_on_sc=False)

f32 = jnp.float32


CPS = 5
SS = CPS * K
NSS = PW // SS
PAIRS = (NSS - 1) // 2


def _gather_body(x_hbm, src_hbm, dst_hbm, gs_hbm, gd_hbm,
                 src_v, dst_v, bufs0, bufd0, bufs1, bufd1,
                 gsem0, gsem1, wsem0, wsem1):
    c = lax.axis_index("c")
    s = lax.axis_index("s")
    wid = s * NC + c
    base = wid * PW
    pltpu.sync_copy(src_hbm.at[pl.ds(base, PW)], src_v)
    pltpu.sync_copy(dst_hbm.at[pl.ds(base, PW)], dst_v)

    def issue_gathers(t, bs, bd, sem):
        for q in range(CPS):
            off = t * SS + q * K
            pltpu.async_copy(
                x_hbm.at[src_v.at[pl.ds(off, K)]], bs.at[pl.ds(q * K, K)], sem)
            pltpu.async_copy(
                x_hbm.at[dst_v.at[pl.ds(off, K)]], bd.at[pl.ds(q * K, K)], sem)

    def drain_gathers(t, bs, bd, sem):
        for q in range(CPS):
            off = t * SS + q * K
            pltpu.make_async_copy(
                x_hbm.at[src_v.at[pl.ds(off, K)]], bs.at[pl.ds(q * K, K)],
                sem).wait()
            pltpu.make_async_copy(
                x_hbm.at[dst_v.at[pl.ds(off, K)]], bd.at[pl.ds(q * K, K)],
                sem).wait()

    def issue_writes(t, bs, bd, sem):
        off = base + t * SS
        pltpu.async_copy(bs, gs_hbm.at[pl.ds(off, SS)], sem)
        pltpu.async_copy(bd, gd_hbm.at[pl.ds(off, SS)], sem)

    def drain_writes(t, bs, bd, sem):
        off = base + t * SS
        pltpu.make_async_copy(bs, gs_hbm.at[pl.ds(off, SS)], sem).wait()
        pltpu.make_async_copy(bd, gd_hbm.at[pl.ds(off, SS)], sem).wait()

    issue_gathers(0, bufs0, bufd0, gsem0)

    def body(t, carry):
        a = 2 * t
        b = 2 * t + 1
        issue_gathers(b, bufs1, bufd1, gsem1)
        drain_gathers(a, bufs0, bufd0, gsem0)
        issue_writes(a, bufs0, bufd0, wsem0)
        drain_gathers(b, bufs1, bufd1, gsem1)
        issue_writes(b, bufs1, bufd1, wsem1)
        drain_writes(a, bufs0, bufd0, wsem0)
        drain_writes(b, bufs1, bufd1, wsem1)
        issue_gathers(a + 2, bufs0, bufd0, gsem0)
        return carry

    lax.fori_loop(0, PAIRS, body, 0)
    t_tail = NSS - 1
    drain_gathers(t_tail, bufs0, bufd0, gsem0)
    issue_writes(t_tail, bufs0, bufd0, wsem0)
    drain_writes(t_tail, bufs0, bufd0, wsem0)


_gather = pl.kernel(
    _gather_body,
    out_type=[jax.ShapeDtypeStruct((E, HID), f32),
              jax.ShapeDtypeStruct((E, HID), f32)],
    mesh=_mesh,
    scratch_types=[
        pltpu.VMEM((PW,), jnp.int32),
        pltpu.VMEM((PW,), jnp.int32),
        pltpu.VMEM((SS, HID), f32),
        pltpu.VMEM((SS, HID), f32),
        pltpu.VMEM((SS, HID), f32),
        pltpu.VMEM((SS, HID), f32),
        pltpu.SemaphoreType.DMA,
        pltpu.SemaphoreType.DMA,
        pltpu.SemaphoreType.DMA,
        pltpu.SemaphoreType.DMA,
    ],
    compiler_params=_sc_params,
)


def _scatter_body(m_hbm, dst2_hbm, agg_hbm, dst_v, mbuf0, mbuf1, agg_sh,
                  rsem0, rsem1, ssem0, ssem1):
    c = lax.axis_index("c")
    s = lax.axis_index("s")
    wid = s * NC + c
    base = wid * PW

    def zbody(r, carry):
        for l in range(HID // 16):
            mbuf0[r, pl.ds(l * 16, 16)] = jnp.zeros((16,), f32)
        return carry
    lax.fori_loop(0, SS, zbody, 0)
    pltpu.sync_copy(mbuf0, agg_sh.at[pl.ds(s * RPT, SS)])
    pltpu.sync_copy(mbuf0.at[pl.ds(0, RPT - SS)],
                    agg_sh.at[pl.ds(s * RPT + SS, RPT - SS)])
    plsc.subcore_barrier()

    pltpu.sync_copy(dst2_hbm.at[pl.ds(wid * NCHUNK, NCHUNK)], dst_v)

    def issue_read(t, mb, sem):
        pltpu.async_copy(m_hbm.at[pl.ds(base + t * SS, SS)], mb, sem)

    def drain_read(t, mb, sem):
        pltpu.make_async_copy(m_hbm.at[pl.ds(base + t * SS, SS)], mb,
                              sem).wait()

    def issue_scatters(t, mb, sem):
        for q in range(CPS):
            pltpu.async_copy(mb.at[pl.ds(q * K, K)],
                             agg_sh.at[dst_v.at[t * CPS + q]], sem, add=True)

    def drain_scatters(t, mb, sem):
        for q in range(CPS):
            pltpu.make_async_copy(mb.at[pl.ds(q * K, K)],
                                  agg_sh.at[dst_v.at[t * CPS + q]],
                                  sem).wait()

    issue_read(0, mbuf0, rsem0)

    def body(t, carry):
        a = 2 * t
        b = 2 * t + 1
        issue_read(b, mbuf1, rsem1)
        drain_read(a, mbuf0, rsem0)
        issue_scatters(a, mbuf0, ssem0)
        drain_read(b, mbuf1, rsem1)
        issue_scatters(b, mbuf1, ssem1)
        drain_scatters(a, mbuf0, ssem0)
        drain_scatters(b, mbuf1, ssem1)
        issue_read(a + 2, mbuf0, rsem0)
        return carry

    lax.fori_loop(0, PAIRS, body, 0)
    t_tail = NSS - 1
    drain_read(t_tail, mbuf0, rsem0)
    issue_scatters(t_tail, mbuf0, ssem0)
    drain_scatters(t_tail, mbuf0, ssem0)
    plsc.subcore_barrier()

    half = RPT // 2
    pltpu.sync_copy(agg_sh.at[pl.ds(s * RPT, half)], mbuf0.at[pl.ds(0, half)])
    pltpu.async_copy(mbuf0.at[pl.ds(0, half)],
                     agg_hbm.at[c, pl.ds(s * RPT, half)], rsem0)
    pltpu.sync_copy(agg_sh.at[pl.ds(s * RPT + half, half)],
                    mbuf1.at[pl.ds(0, half)])
    pltpu.async_copy(mbuf1.at[pl.ds(0, half)],
                     agg_hbm.at[c, pl.ds(s * RPT + half, half)], rsem1)
    pltpu.make_async_copy(mbuf0.at[pl.ds(0, half)],
                          agg_hbm.at[c, pl.ds(s * RPT, half)], rsem0).wait()
    pltpu.make_async_copy(mbuf1.at[pl.ds(0, half)],
                          agg_hbm.at[c, pl.ds(s * RPT + half, half)],
                          rsem1).wait()


_scatter = pl.kernel(
    _scatter_body,
    out_type=jax.ShapeDtypeStruct((NC, NPAD, HID), f32),
    mesh=_mesh,
    scratch_types=[
        pltpu.VMEM((NCHUNK, K), jnp.int32),
        pltpu.VMEM((SS, HID), f32),
        pltpu.VMEM((SS, HID), f32),
        pltpu.VMEM_SHARED((NPAD, HID), f32),
        pltpu.SemaphoreType.DMA,
        pltpu.SemaphoreType.DMA,
        pltpu.SemaphoreType.DMA,
        pltpu.SemaphoreType.DMA,
    ],
    compiler_params=_sc_params,
)


def _encoder_body(h_ref, w_ref, b_ref, out_ref):
    out_ref[...] = h_ref[...] @ w_ref[...] + b_ref[...]


def _edge_mlp_body(gs_ref, gd_ref, attr_ref, ws_ref, wd_ref, we_ref,
                   b1_ref, w2_ref, b2_ref, out_ref):
    z = (gs_ref[...] @ ws_ref[...] + gd_ref[...] @ wd_ref[...]
         + attr_ref[...] @ we_ref[...] + b1_ref[...])
    m1 = jnp.maximum(z, 0.0)
    out_ref[...] = jnp.maximum(m1 @ w2_ref[...] + b2_ref[...], 0.0)


def _update_body(x_ref, agg_ref, w1x_ref, w1a_ref, b1_ref, w2_ref, b2_ref,
                 out_ref):
    agg = agg_ref[0] + agg_ref[1]
    z = jnp.maximum(
        x_ref[...] @ w1x_ref[...] + agg @ w1a_ref[...] + b1_ref[...], 0.0)
    out_ref[...] = x_ref[...] + (z @ w2_ref[...] + b2_ref[...])


def _decoder_body(x_ref, w1_ref, b1_ref, w2_ref, b2_ref, out_ref):
    d = jnp.maximum(x_ref[...] @ w1_ref[...] + b1_ref[...], 0.0)
    out_ref[...] = d @ w2_ref[...] + b2_ref[...]


EB = 2000


def _full(shape):
    return pl.BlockSpec(shape, lambda *_: tuple(0 for _ in shape))


def _encoder(h, w, b):
    return pl.pallas_call(
        _encoder_body,
        out_shape=jax.ShapeDtypeStruct((N, HID), f32),
        in_specs=[_full((N, NODE_IN)), _full((NODE_IN, HID)), _full((1, HID))],
        out_specs=_full((N, HID)),
    )(h, w, b)


def _edge_mlp(gs, gd, attr, ws, wd, we, b1, w2, b2):
    grid = (E // EB,)
    eb = pl.BlockSpec((EB, HID), lambda i: (i, 0))
    ea = pl.BlockSpec((EB, EDGE_DIM), lambda i: (i, 0))
    wspec = pl.BlockSpec((HID, HID), lambda i: (0, 0))
    wespec = pl.BlockSpec((EDGE_DIM, HID), lambda i: (0, 0))
    bspec = pl.BlockSpec((1, HID), lambda i: (0, 0))
    return pl.pallas_call(
        _edge_mlp_body,
        grid=grid,
        out_shape=jax.ShapeDtypeStruct((E, HID), f32),
        in_specs=[eb, eb, ea, wspec, wspec, wespec, bspec, wspec, bspec],
        out_specs=eb,
    )(gs, gd, attr, ws, wd, we, b1, w2, b2)


def _update(x, agg, w1x, w1a, b1, w2, b2):
    return pl.pallas_call(
        _update_body,
        out_shape=jax.ShapeDtypeStruct((N, HID), f32),
        in_specs=[_full((N, HID)), _full((NC, N, HID)), _full((HID, HID)),
                  _full((HID, HID)), _full((1, HID)), _full((HID, HID)),
                  _full((1, HID))],
        out_specs=_full((N, HID)),
    )(x, agg, w1x, w1a, b1, w2, b2)


def _decoder(x, w1, b1, w2, b2):
    return pl.pallas_call(
        _decoder_body,
        out_shape=jax.ShapeDtypeStruct((N, 1), f32),
        in_specs=[_full((N, HID)), _full((HID, HID)), _full((1, HID)),
                  _full((HID, 1)), _full((1, 1))],
        out_specs=_full((N, 1)),
    )(x, w1, b1, w2, b2)


def kernel(h, edge_index, edge_attr, params):
    src = edge_index[0]
    dst = edge_index[1]
    dst2 = dst.reshape(E // K, K)

    x = _encoder(h, params["enc_w"], params["enc_b"].reshape(1, HID))
    for p in params["layers"]:
        ws = p["m_w1"][:HID]
        wd = p["m_w1"][HID:2 * HID]
        we = p["m_w1"][2 * HID:]
        gs, gd = _gather(x, src, dst)
        m = _edge_mlp(gs, gd, edge_attr, ws, wd, we,
                      p["m_b1"].reshape(1, HID), p["m_w2"],
                      p["m_b2"].reshape(1, HID))
        aggp = _scatter(m, dst2)
        agg = aggp[:, :N, :]
        x = _update(x, agg, p["u_w1"][:HID], p["u_w1"][HID:],
                    p["u_b1"].reshape(1, HID), p["u_w2"],
                    p["u_b2"].reshape(1, HID))
    psi = _decoder(x, params["dec_w1"], params["dec_b1"].reshape(1, HID),
                   params["dec_w2"], params["dec_b2"].reshape(1, 1))
    return psi


# speedup vs baseline: 2.3894x; 1.0765x over previous
import functools

import jax
import jax.numpy as jnp
from jax import lax
from jax.experimental import pallas as pl
from jax.experimental.pallas import tpu as pltpu
from jax.experimental.pallas import tpu_sc as plsc

N = 10000
E = 320000
HID = 64
NODE_IN = 5
EDGE_DIM = 3

NC = 2
NS = 16
NW = NC * NS
PW = E // NW
K = 80
NCHUNK = PW // K
NPAD = 10240
RPT = NPAD // NS

_mesh = plsc.VectorSubcoreMesh(
    core_axis_name="c", subcore_axis_name="s", num_cores=NC, num_subcores=NS)

_sc_params = pltpu.CompilerParams(use_tc_tiling_on_sc=False)

f32 = jnp.float32


CPS = 5
SS = CPS * K
NSS = PW // SS
PAIRS = (NSS - 1) // 2


def _gather_body(a_hbm, b_hbm, src_hbm, dst_hbm, gab_hbm,
                 src_v, dst_v, bufs0, bufd0, bufs1, bufd1,
                 gsem0, gsem1, wsem0, wsem1):
    c = lax.axis_index("c")
    s = lax.axis_index("s")
    wid = s * NC + c
    base = wid * PW
    pltpu.sync_copy(src_hbm.at[pl.ds(base, PW)], src_v)
    pltpu.sync_copy(dst_hbm.at[pl.ds(base, PW)], dst_v)

    def issue_gathers(t, bs, bd, sem):
        for q in range(CPS):
            off = t * SS + q * K
            pltpu.async_copy(
                a_hbm.at[src_v.at[pl.ds(off, K)]], bs.at[pl.ds(q * K, K)], sem)
            pltpu.async_copy(
                b_hbm.at[dst_v.at[pl.ds(off, K)]], bd.at[pl.ds(q * K, K)], sem)

    def drain_gathers(t, bs, bd, sem):
        for q in range(CPS):
            off = t * SS + q * K
            pltpu.make_async_copy(
                a_hbm.at[src_v.at[pl.ds(off, K)]], bs.at[pl.ds(q * K, K)],
                sem).wait()
            pltpu.make_async_copy(
                b_hbm.at[dst_v.at[pl.ds(off, K)]], bd.at[pl.ds(q * K, K)],
                sem).wait()

    def issue_writes(t, bs, bd, sem):
        off = base + t * SS
        pltpu.async_copy(bs, gab_hbm.at[0, pl.ds(off, SS)], sem)
        pltpu.async_copy(bd, gab_hbm.at[1, pl.ds(off, SS)], sem)

    def drain_writes(t, bs, bd, sem):
        off = base + t * SS
        pltpu.make_async_copy(bs, gab_hbm.at[0, pl.ds(off, SS)], sem).wait()
        pltpu.make_async_copy(bd, gab_hbm.at[1, pl.ds(off, SS)], sem).wait()

    issue_gathers(0, bufs0, bufd0, gsem0)

    def body(t, carry):
        a = 2 * t
        b = 2 * t + 1
        issue_gathers(b, bufs1, bufd1, gsem1)
        drain_gathers(a, bufs0, bufd0, gsem0)
        issue_writes(a, bufs0, bufd0, wsem0)
        drain_gathers(b, bufs1, bufd1, gsem1)
        issue_writes(b, bufs1, bufd1, wsem1)
        drain_writes(a, bufs0, bufd0, wsem0)
        drain_writes(b, bufs1, bufd1, wsem1)
        issue_gathers(a + 2, bufs0, bufd0, gsem0)
        return carry

    lax.fori_loop(0, PAIRS, body, 0)
    t_tail = NSS - 1
    drain_gathers(t_tail, bufs0, bufd0, gsem0)
    issue_writes(t_tail, bufs0, bufd0, wsem0)
    drain_writes(t_tail, bufs0, bufd0, wsem0)


_gather = pl.kernel(
    _gather_body,
    out_type=jax.ShapeDtypeStruct((2, E, HID), f32),
    mesh=_mesh,
    scratch_types=[
        pltpu.VMEM((PW,), jnp.int32),
        pltpu.VMEM((PW,), jnp.int32),
        pltpu.VMEM((SS, HID), f32),
        pltpu.VMEM((SS, HID), f32),
        pltpu.VMEM((SS, HID), f32),
        pltpu.VMEM((SS, HID), f32),
        pltpu.SemaphoreType.DMA,
        pltpu.SemaphoreType.DMA,
        pltpu.SemaphoreType.DMA,
        pltpu.SemaphoreType.DMA,
    ],
    compiler_params=_sc_params,
)


def _scatter_body(m_hbm, dst2_hbm, agg_hbm, dst_v, mbuf0, mbuf1, agg_sh,
                  rsem0, rsem1, ssem0, ssem1):
    c = lax.axis_index("c")
    s = lax.axis_index("s")
    wid = s * NC + c
    base = wid * PW

    def zbody(r, carry):
        for l in range(HID // 16):
            mbuf0[r, pl.ds(l * 16, 16)] = jnp.zeros((16,), f32)
        return carry
    lax.fori_loop(0, SS, zbody, 0)
    pltpu.sync_copy(mbuf0, agg_sh.at[pl.ds(s * RPT, SS)])
    pltpu.sync_copy(mbuf0.at[pl.ds(0, RPT - SS)],
                    agg_sh.at[pl.ds(s * RPT + SS, RPT - SS)])
    plsc.subcore_barrier()

    pltpu.sync_copy(dst2_hbm.at[pl.ds(wid * NCHUNK, NCHUNK)], dst_v)

    def issue_read(t, mb, sem):
        pltpu.async_copy(m_hbm.at[pl.ds(base + t * SS, SS)], mb, sem)

    def drain_read(t, mb, sem):
        pltpu.make_async_copy(m_hbm.at[pl.ds(base + t * SS, SS)], mb,
                              sem).wait()

    def issue_scatters(t, mb, sem):
        for q in range(CPS):
            pltpu.async_copy(mb.at[pl.ds(q * K, K)],
                             agg_sh.at[dst_v.at[t * CPS + q]], sem, add=True)

    def drain_scatters(t, mb, sem):
        for q in range(CPS):
            pltpu.make_async_copy(mb.at[pl.ds(q * K, K)],
                                  agg_sh.at[dst_v.at[t * CPS + q]],
                                  sem).wait()

    issue_read(0, mbuf0, rsem0)

    def body(t, carry):
        a = 2 * t
        b = 2 * t + 1
        issue_read(b, mbuf1, rsem1)
        drain_read(a, mbuf0, rsem0)
        issue_scatters(a, mbuf0, ssem0)
        drain_read(b, mbuf1, rsem1)
        issue_scatters(b, mbuf1, ssem1)
        drain_scatters(a, mbuf0, ssem0)
        drain_scatters(b, mbuf1, ssem1)
        issue_read(a + 2, mbuf0, rsem0)
        return carry

    lax.fori_loop(0, PAIRS, body, 0)
    t_tail = NSS - 1
    drain_read(t_tail, mbuf0, rsem0)
    issue_scatters(t_tail, mbuf0, ssem0)
    drain_scatters(t_tail, mbuf0, ssem0)
    plsc.subcore_barrier()

    half = RPT // 2
    pltpu.sync_copy(agg_sh.at[pl.ds(s * RPT, half)], mbuf0.at[pl.ds(0, half)])
    pltpu.async_copy(mbuf0.at[pl.ds(0, half)],
                     agg_hbm.at[c, pl.ds(s * RPT, half)], rsem0)
    pltpu.sync_copy(agg_sh.at[pl.ds(s * RPT + half, half)],
                    mbuf1.at[pl.ds(0, half)])
    pltpu.async_copy(mbuf1.at[pl.ds(0, half)],
                     agg_hbm.at[c, pl.ds(s * RPT + half, half)], rsem1)
    pltpu.make_async_copy(mbuf0.at[pl.ds(0, half)],
                          agg_hbm.at[c, pl.ds(s * RPT, half)], rsem0).wait()
    pltpu.make_async_copy(mbuf1.at[pl.ds(0, half)],
                          agg_hbm.at[c, pl.ds(s * RPT + half, half)],
                          rsem1).wait()


_scatter = pl.kernel(
    _scatter_body,
    out_type=jax.ShapeDtypeStruct((NC, NPAD, HID), f32),
    mesh=_mesh,
    scratch_types=[
        pltpu.VMEM((NCHUNK, K), jnp.int32),
        pltpu.VMEM((SS, HID), f32),
        pltpu.VMEM((SS, HID), f32),
        pltpu.VMEM_SHARED((NPAD, HID), f32),
        pltpu.SemaphoreType.DMA,
        pltpu.SemaphoreType.DMA,
        pltpu.SemaphoreType.DMA,
        pltpu.SemaphoreType.DMA,
    ],
    compiler_params=_sc_params,
)


def _encoder_body(h_ref, w_ref, b_ref, ws_ref, wd_ref,
                  x_ref, a_ref, b2_ref):
    x = h_ref[...] @ w_ref[...] + b_ref[...]
    x_ref[...] = x
    a_ref[...] = x @ ws_ref[...]
    b2_ref[...] = x @ wd_ref[...]


def _edge_mlp_body(gab_ref, attr_ref, we_ref, b1_ref, w2_ref, b2_ref,
                   out_ref):
    z = (gab_ref[0] + gab_ref[1]
         + attr_ref[...] @ we_ref[...] + b1_ref[...])
    m1 = jnp.maximum(z, 0.0)
    out_ref[...] = jnp.maximum(m1 @ w2_ref[...] + b2_ref[...], 0.0)


def _update_body(x_ref, agg_ref, w1x_ref, w1a_ref, b1_ref, w2_ref, b2_ref,
                 ws_ref, wd_ref, out_ref, a_ref, bt_ref):
    agg = agg_ref[0] + agg_ref[1]
    z = jnp.maximum(
        x_ref[...] @ w1x_ref[...] + agg @ w1a_ref[...] + b1_ref[...], 0.0)
    xn = x_ref[...] + (z @ w2_ref[...] + b2_ref[...])
    out_ref[...] = xn
    a_ref[...] = xn @ ws_ref[...]
    bt_ref[...] = xn @ wd_ref[...]


def _update_last_body(x_ref, agg_ref, w1x_ref, w1a_ref, b1_ref, w2_ref,
                      b2_ref, dw1_ref, db1_ref, dw2_ref, db2_ref, out_ref):
    agg = agg_ref[0] + agg_ref[1]
    z = jnp.maximum(
        x_ref[...] @ w1x_ref[...] + agg @ w1a_ref[...] + b1_ref[...], 0.0)
    xn = x_ref[...] + (z @ w2_ref[...] + b2_ref[...])
    d = jnp.maximum(xn @ dw1_ref[...] + db1_ref[...], 0.0)
    out_ref[...] = d @ dw2_ref[...] + db2_ref[...]


EB = 4000


def _full(shape):
    return pl.BlockSpec(shape, lambda *_: tuple(0 for _ in shape))


def _encoder(h, w, b, ws, wd):
    out = jax.ShapeDtypeStruct((N, HID), f32)
    return pl.pallas_call(
        _encoder_body,
        out_shape=[out, out, out],
        in_specs=[_full((N, NODE_IN)), _full((NODE_IN, HID)), _full((1, HID)),
                  _full((HID, HID)), _full((HID, HID))],
        out_specs=[_full((N, HID))] * 3,
    )(h, w, b, ws, wd)


def _edge_mlp(gab, attr, we, b1, w2, b2):
    grid = (E // EB,)
    g2 = pl.BlockSpec((2, EB, HID), lambda i: (0, i, 0))
    eb = pl.BlockSpec((EB, HID), lambda i: (i, 0))
    ea = pl.BlockSpec((EB, EDGE_DIM), lambda i: (i, 0))
    wspec = pl.BlockSpec((HID, HID), lambda i: (0, 0))
    wespec = pl.BlockSpec((EDGE_DIM, HID), lambda i: (0, 0))
    bspec = pl.BlockSpec((1, HID), lambda i: (0, 0))
    return pl.pallas_call(
        _edge_mlp_body,
        grid=grid,
        out_shape=jax.ShapeDtypeStruct((E, HID), f32),
        in_specs=[g2, ea, wespec, bspec, wspec, bspec],
        out_specs=eb,
    )(gab, attr, we, b1, w2, b2)


def _update(x, agg, w1x, w1a, b1, w2, b2, ws, wd):
    out = jax.ShapeDtypeStruct((N, HID), f32)
    return pl.pallas_call(
        _update_body,
        out_shape=[out, out, out],
        in_specs=[_full((N, HID)), _full((NC, N, HID)), _full((HID, HID)),
                  _full((HID, HID)), _full((1, HID)), _full((HID, HID)),
                  _full((1, HID)), _full((HID, HID)), _full((HID, HID))],
        out_specs=[_full((N, HID))] * 3,
    )(x, agg, w1x, w1a, b1, w2, b2, ws, wd)


def _update_last(x, agg, w1x, w1a, b1, w2, b2, dw1, db1, dw2, db2):
    return pl.pallas_call(
        _update_last_body,
        out_shape=jax.ShapeDtypeStruct((N, 1), f32),
        in_specs=[_full((N, HID)), _full((NC, N, HID)), _full((HID, HID)),
                  _full((HID, HID)), _full((1, HID)), _full((HID, HID)),
                  _full((1, HID)), _full((HID, HID)), _full((1, HID)),
                  _full((HID, 1)), _full((1, 1))],
        out_specs=_full((N, 1)),
    )(x, agg, w1x, w1a, b1, w2, b2, dw1, db1, dw2, db2)


def kernel(h, edge_index, edge_attr, params):
    src = edge_index[0]
    dst = edge_index[1]
    dst2 = dst.reshape(E // K, K)
    layers = params["layers"]

    def mw(p):
        return (p["m_w1"][:HID], p["m_w1"][HID:2 * HID], p["m_w1"][2 * HID:])

    ws0, wd0, _ = mw(layers[0])
    x, a, bt = _encoder(h, params["enc_w"], params["enc_b"].reshape(1, HID),
                        ws0, wd0)
    for li, p in enumerate(layers):
        _, _, we = mw(p)
        gab = _gather(a, bt, src, dst)
        m = _edge_mlp(gab, edge_attr, we, p["m_b1"].reshape(1, HID),
                      p["m_w2"], p["m_b2"].reshape(1, HID))
        aggp = _scatter(m, dst2)
        agg = aggp[:, :N, :]
        uargs = (x, agg, p["u_w1"][:HID], p["u_w1"][HID:],
                 p["u_b1"].reshape(1, HID), p["u_w2"],
                 p["u_b2"].reshape(1, HID))
        if li + 1 < len(layers):
            wsn, wdn, _ = mw(layers[li + 1])
            x, a, bt = _update(*uargs, wsn, wdn)
        else:
            psi = _update_last(*uargs, params["dec_w1"],
                               params["dec_b1"].reshape(1, HID),
                               params["dec_w2"],
                               params["dec_b2"].reshape(1, 1))
    return psi


# speedup vs baseline: 4.8581x; 2.0332x over previous
import functools

import jax
import jax.numpy as jnp
from jax import lax
from jax.experimental import pallas as pl
from jax.experimental.pallas import tpu as pltpu
from jax.experimental.pallas import tpu_sc as plsc

N = 10000
E = 320000
HID = 64
NODE_IN = 5
EDGE_DIM = 3

NC = 2
NS = 16
NW = NC * NS
PW = E // NW
K = 80
NCHUNK = PW // K
NPAD = 10240
RPT = NPAD // NS

_mesh = plsc.VectorSubcoreMesh(
    core_axis_name="c", subcore_axis_name="s", num_cores=NC, num_subcores=NS)

_sc_params = pltpu.CompilerParams(use_tc_tiling_on_sc=False)

f32 = jnp.float32


CPS = 5
SS = CPS * K
NSS = PW // SS
PAIRS = (NSS - 1) // 2


def _gather_body(a_hbm, b_hbm, src_hbm, dst_hbm, gab_hbm,
                 src_v, dst_v, bufs0, bufd0, bufs1, bufd1,
                 gsem0, gsem1, wsem0, wsem1):
    c = lax.axis_index("c")
    s = lax.axis_index("s")
    wid = s * NC + c
    base = wid * PW
    pltpu.sync_copy(src_hbm.at[pl.ds(base, PW)], src_v)
    pltpu.sync_copy(dst_hbm.at[pl.ds(base, PW)], dst_v)

    def issue_gathers(t, bs, bd, sem):
        for q in range(CPS):
            off = t * SS + q * K
            pltpu.async_copy(
                a_hbm.at[src_v.at[pl.ds(off, K)]], bs.at[pl.ds(q * K, K)], sem)
            pltpu.async_copy(
                b_hbm.at[dst_v.at[pl.ds(off, K)]], bd.at[pl.ds(q * K, K)], sem)

    def drain_gathers(t, bs, bd, sem):
        for q in range(CPS):
            off = t * SS + q * K
            pltpu.make_async_copy(
                a_hbm.at[src_v.at[pl.ds(off, K)]], bs.at[pl.ds(q * K, K)],
                sem).wait()
            pltpu.make_async_copy(
                b_hbm.at[dst_v.at[pl.ds(off, K)]], bd.at[pl.ds(q * K, K)],
                sem).wait()

    def issue_writes(t, bs, bd, sem):
        off = base + t * SS
        pltpu.async_copy(bs, gab_hbm.at[pl.ds(off, SS), pl.ds(0, HID)], sem)
        pltpu.async_copy(bd, gab_hbm.at[pl.ds(off, SS), pl.ds(HID, HID)], sem)

    def drain_writes(t, bs, bd, sem):
        off = base + t * SS
        pltpu.make_async_copy(
            bs, gab_hbm.at[pl.ds(off, SS), pl.ds(0, HID)], sem).wait()
        pltpu.make_async_copy(
            bd, gab_hbm.at[pl.ds(off, SS), pl.ds(HID, HID)], sem).wait()

    issue_gathers(0, bufs0, bufd0, gsem0)

    def body(t, carry):
        a = 2 * t
        b = 2 * t + 1
        issue_gathers(b, bufs1, bufd1, gsem1)
        drain_gathers(a, bufs0, bufd0, gsem0)
        issue_writes(a, bufs0, bufd0, wsem0)
        drain_gathers(b, bufs1, bufd1, gsem1)
        issue_writes(b, bufs1, bufd1, wsem1)
        drain_writes(a, bufs0, bufd0, wsem0)
        drain_writes(b, bufs1, bufd1, wsem1)
        issue_gathers(a + 2, bufs0, bufd0, gsem0)
        return carry

    lax.fori_loop(0, PAIRS, body, 0)
    t_tail = NSS - 1
    drain_gathers(t_tail, bufs0, bufd0, gsem0)
    issue_writes(t_tail, bufs0, bufd0, wsem0)
    drain_writes(t_tail, bufs0, bufd0, wsem0)


_gather = pl.kernel(
    _gather_body,
    out_type=jax.ShapeDtypeStruct((E, 2 * HID), f32),
    mesh=_mesh,
    scratch_types=[
        pltpu.VMEM((PW,), jnp.int32),
        pltpu.VMEM((PW,), jnp.int32),
        pltpu.VMEM((SS, HID), f32),
        pltpu.VMEM((SS, HID), f32),
        pltpu.VMEM((SS, HID), f32),
        pltpu.VMEM((SS, HID), f32),
        pltpu.SemaphoreType.DMA,
        pltpu.SemaphoreType.DMA,
        pltpu.SemaphoreType.DMA,
        pltpu.SemaphoreType.DMA,
    ],
    compiler_params=_sc_params,
)


def _scatter_body(m_hbm, dst2_hbm, agg_hbm, dst_v, mbuf0, mbuf1, agg_sh,
                  rsem0, rsem1, ssem0, ssem1):
    c = lax.axis_index("c")
    s = lax.axis_index("s")
    wid = s * NC + c
    base = wid * PW

    def zbody(r, carry):
        for l in range(HID // 16):
            mbuf0[r, pl.ds(l * 16, 16)] = jnp.zeros((16,), f32)
        return carry
    lax.fori_loop(0, SS, zbody, 0)
    pltpu.sync_copy(mbuf0, agg_sh.at[pl.ds(s * RPT, SS)])
    pltpu.sync_copy(mbuf0.at[pl.ds(0, RPT - SS)],
                    agg_sh.at[pl.ds(s * RPT + SS, RPT - SS)])
    plsc.subcore_barrier()

    pltpu.sync_copy(dst2_hbm.at[pl.ds(wid * NCHUNK, NCHUNK)], dst_v)

    def issue_read(t, mb, sem):
        pltpu.async_copy(
            m_hbm.at[pl.ds(base + t * SS, SS), pl.ds(0, HID)], mb, sem)

    def drain_read(t, mb, sem):
        pltpu.make_async_copy(
            m_hbm.at[pl.ds(base + t * SS, SS), pl.ds(0, HID)], mb, sem).wait()

    def issue_scatters(t, mb, sem):
        for q in range(CPS):
            pltpu.async_copy(mb.at[pl.ds(q * K, K)],
                             agg_sh.at[dst_v.at[t * CPS + q]], sem, add=True)

    def drain_scatters(t, mb, sem):
        for q in range(CPS):
            pltpu.make_async_copy(mb.at[pl.ds(q * K, K)],
                                  agg_sh.at[dst_v.at[t * CPS + q]],
                                  sem).wait()

    issue_read(0, mbuf0, rsem0)

    def body(t, carry):
        a = 2 * t
        b = 2 * t + 1
        issue_read(b, mbuf1, rsem1)
        drain_read(a, mbuf0, rsem0)
        issue_scatters(a, mbuf0, ssem0)
        drain_read(b, mbuf1, rsem1)
        issue_scatters(b, mbuf1, ssem1)
        drain_scatters(a, mbuf0, ssem0)
        drain_scatters(b, mbuf1, ssem1)
        issue_read(a + 2, mbuf0, rsem0)
        return carry

    lax.fori_loop(0, PAIRS, body, 0)
    t_tail = NSS - 1
    drain_read(t_tail, mbuf0, rsem0)
    issue_scatters(t_tail, mbuf0, ssem0)
    drain_scatters(t_tail, mbuf0, ssem0)
    plsc.subcore_barrier()

    half = RPT // 2
    pltpu.sync_copy(agg_sh.at[pl.ds(s * RPT, half)], mbuf0.at[pl.ds(0, half)])
    pltpu.async_copy(mbuf0.at[pl.ds(0, half)],
                     agg_hbm.at[c, pl.ds(s * RPT, half)], rsem0)
    pltpu.sync_copy(agg_sh.at[pl.ds(s * RPT + half, half)],
                    mbuf1.at[pl.ds(0, half)])
    pltpu.async_copy(mbuf1.at[pl.ds(0, half)],
                     agg_hbm.at[c, pl.ds(s * RPT + half, half)], rsem1)
    pltpu.make_async_copy(mbuf0.at[pl.ds(0, half)],
                          agg_hbm.at[c, pl.ds(s * RPT, half)], rsem0).wait()
    pltpu.make_async_copy(mbuf1.at[pl.ds(0, half)],
                          agg_hbm.at[c, pl.ds(s * RPT + half, half)],
                          rsem1).wait()


_scatter = pl.kernel(
    _scatter_body,
    out_type=jax.ShapeDtypeStruct((NC, NPAD, HID), f32),
    mesh=_mesh,
    scratch_types=[
        pltpu.VMEM((NCHUNK, K), jnp.int32),
        pltpu.VMEM((SS, HID), f32),
        pltpu.VMEM((SS, HID), f32),
        pltpu.VMEM_SHARED((NPAD, HID), f32),
        pltpu.SemaphoreType.DMA,
        pltpu.SemaphoreType.DMA,
        pltpu.SemaphoreType.DMA,
        pltpu.SemaphoreType.DMA,
    ],
    compiler_params=_sc_params,
)


def _encoder_body(h_ref, w_ref, b_ref, ws_ref, wd_ref,
                  x_ref, a_ref, b2_ref):
    x = h_ref[...] @ w_ref[...] + b_ref[...]
    x_ref[...] = x
    a_ref[...] = x @ ws_ref[...]
    b2_ref[...] = x @ wd_ref[...]


def _edge_mlp_body(gab_ref, attr_ref, we_ref, b1_ref, w2_ref, b2_ref,
                   out_ref):
    z = (gab_ref[:, :HID] + gab_ref[:, HID:]
         + attr_ref[...] @ we_ref[...] + b1_ref[...])
    m1 = jnp.maximum(z, 0.0)
    out_ref[:, :HID] = jnp.maximum(m1 @ w2_ref[...] + b2_ref[...], 0.0)
    out_ref[:, HID:] = jnp.zeros((EB, HID), f32)


def _update_body(x_ref, agg_ref, w1x_ref, w1a_ref, b1_ref, w2_ref, b2_ref,
                 ws_ref, wd_ref, out_ref, a_ref, bt_ref):
    agg = agg_ref[0] + agg_ref[1]
    z = jnp.maximum(
        x_ref[...] @ w1x_ref[...] + agg @ w1a_ref[...] + b1_ref[...], 0.0)
    xn = x_ref[...] + (z @ w2_ref[...] + b2_ref[...])
    out_ref[...] = xn
    a_ref[...] = xn @ ws_ref[...]
    bt_ref[...] = xn @ wd_ref[...]


def _update_last_body(x_ref, agg_ref, w1x_ref, w1a_ref, b1_ref, w2_ref,
                      b2_ref, dw1_ref, db1_ref, dw2_ref, db2_ref, out_ref):
    agg = agg_ref[0] + agg_ref[1]
    z = jnp.maximum(
        x_ref[...] @ w1x_ref[...] + agg @ w1a_ref[...] + b1_ref[...], 0.0)
    xn = x_ref[...] + (z @ w2_ref[...] + b2_ref[...])
    d = jnp.maximum(xn @ dw1_ref[...] + db1_ref[...], 0.0)
    out_ref[...] = d @ dw2_ref[...] + db2_ref[...]


EB = 4000


def _full(shape):
    return pl.BlockSpec(shape, lambda *_: tuple(0 for _ in shape))


def _encoder(h, w, b, ws, wd):
    out = jax.ShapeDtypeStruct((N, HID), f32)
    return pl.pallas_call(
        _encoder_body,
        out_shape=[out, out, out],
        in_specs=[_full((N, NODE_IN)), _full((NODE_IN, HID)), _full((1, HID)),
                  _full((HID, HID)), _full((HID, HID))],
        out_specs=[_full((N, HID))] * 3,
    )(h, w, b, ws, wd)


def _edge_mlp(gab, attr, we, b1, w2, b2):
    grid = (E // EB,)
    eb = pl.BlockSpec((EB, 2 * HID), lambda i: (i, 0))
    ea = pl.BlockSpec((EB, EDGE_DIM), lambda i: (i, 0))
    wspec = pl.BlockSpec((HID, HID), lambda i: (0, 0))
    wespec = pl.BlockSpec((EDGE_DIM, HID), lambda i: (0, 0))
    bspec = pl.BlockSpec((1, HID), lambda i: (0, 0))
    return pl.pallas_call(
        _edge_mlp_body,
        grid=grid,
        out_shape=jax.ShapeDtypeStruct((E, 2 * HID), f32),
        in_specs=[eb, ea, wespec, bspec, wspec, bspec],
        out_specs=eb,
    )(gab, attr, we, b1, w2, b2)


def _update(x, agg, w1x, w1a, b1, w2, b2, ws, wd):
    out = jax.ShapeDtypeStruct((N, HID), f32)
    return pl.pallas_call(
        _update_body,
        grid=(1,),
        out_shape=[out, out, out],
        in_specs=[_full((N, HID)), _full((NC, N, HID)), _full((HID, HID)),
                  _full((HID, HID)), _full((1, HID)), _full((HID, HID)),
                  _full((1, HID)), _full((HID, HID)), _full((HID, HID))],
        out_specs=[_full((N, HID))] * 3,
    )(x, agg, w1x, w1a, b1, w2, b2, ws, wd)


def _update_last(x, agg, w1x, w1a, b1, w2, b2, dw1, db1, dw2, db2):
    return pl.pallas_call(
        _update_last_body,
        grid=(1,),
        out_shape=jax.ShapeDtypeStruct((N, 1), f32),
        in_specs=[_full((N, HID)), _full((NC, N, HID)), _full((HID, HID)),
                  _full((HID, HID)), _full((1, HID)), _full((HID, HID)),
                  _full((1, HID)), _full((HID, HID)), _full((1, HID)),
                  _full((HID, 1)), _full((1, 1))],
        out_specs=_full((N, 1)),
    )(x, agg, w1x, w1a, b1, w2, b2, dw1, db1, dw2, db2)


def kernel(h, edge_index, edge_attr, params):
    src = edge_index[0]
    dst = edge_index[1]
    dst2 = dst.reshape(E // K, K)
    layers = params["layers"]

    def mw(p):
        return (p["m_w1"][:HID], p["m_w1"][HID:2 * HID], p["m_w1"][2 * HID:])

    ws0, wd0, _ = mw(layers[0])
    x, a, bt = _encoder(h, params["enc_w"], params["enc_b"].reshape(1, HID),
                        ws0, wd0)
    for li, p in enumerate(layers):
        _, _, we = mw(p)
        gab = _gather(a, bt, src, dst)
        m = _edge_mlp(gab, edge_attr, we, p["m_b1"].reshape(1, HID),
                      p["m_w2"], p["m_b2"].reshape(1, HID))
        aggp = _scatter(m, dst2)
        uargs = (x, aggp, p["u_w1"][:HID], p["u_w1"][HID:],
                 p["u_b1"].reshape(1, HID), p["u_w2"],
                 p["u_b2"].reshape(1, HID))
        if li + 1 < len(layers):
            wsn, wdn, _ = mw(layers[li + 1])
            x, a, bt = _update(*uargs, wsn, wdn)
        else:
            psi = _update_last(*uargs, params["dec_w1"],
                               params["dec_b1"].reshape(1, HID),
                               params["dec_w2"],
                               params["dec_b2"].reshape(1, 1))
    return psi
